# baseline (speedup 1.0000x reference)
import functools

import jax
import jax.numpy as jnp
from jax import lax
from jax.experimental import pallas as pl
from jax.experimental.pallas import tpu as pltpu
from jax.experimental.pallas import tpu_sc as plsc

_NC = 2
_NS = 16
_NW = _NC * _NS


@functools.partial(jax.jit, static_argnums=(2,))
def _gather(vectors, flat_idx, chunk):
    B = flat_idx.shape[0]
    D = vectors.shape[1]
    b_per_w = B // _NW
    n_chunks = b_per_w // chunk
    n_pairs = n_chunks // 2
    mesh = plsc.VectorSubcoreMesh(core_axis_name="c", subcore_axis_name="s")

    @functools.partial(
        pl.kernel,
        mesh=mesh,
        out_type=jax.ShapeDtypeStruct((B, D), jnp.float32),
        scratch_types=[
            pltpu.VMEM((chunk,), jnp.int32),
            pltpu.VMEM((chunk,), jnp.int32),
            pltpu.VMEM((chunk, D), jnp.float32),
            pltpu.VMEM((chunk, D), jnp.float32),
            pltpu.SemaphoreType.DMA,
            pltpu.SemaphoreType.DMA,
            pltpu.SemaphoreType.DMA,
            pltpu.SemaphoreType.DMA,
            pltpu.SemaphoreType.DMA,
            pltpu.SemaphoreType.DMA,
        ],
        compiler_params=pltpu.CompilerParams(use_tc_tiling_on_sc=False),
    )
    def k(table_hbm, idx_hbm, out_hbm, idx0, idx1, rows0, rows1,
          is0, is1, gs0, gs1, os0, os1):
        idxs, rows = (idx0, idx1), (rows0, rows1)
        isem, gsem, osem = (is0, is1), (gs0, gs1), (os0, os1)
        wid = lax.axis_index("s") * _NC + lax.axis_index("c")
        base = wid * b_per_w

        def fire_idx(g, b):
            pltpu.async_copy(
                idx_hbm.at[pl.ds(base + g * chunk, chunk)], idxs[b], isem[b])

        def wait_idx(b):
            pltpu.make_async_copy(
                idx_hbm.at[pl.ds(base, chunk)], idxs[b], isem[b]).wait()

        def gather(b):
            pltpu.async_copy(table_hbm.at[idxs[b]], rows[b], gsem[b]).wait()

        def fire_out(g, b):
            pltpu.async_copy(
                rows[b], out_hbm.at[pl.ds(base + g * chunk, chunk)], osem[b])

        def wait_out(b):
            pltpu.make_async_copy(
                rows[b], out_hbm.at[pl.ds(base, chunk)], osem[b]).wait()

        fire_idx(0, 0)
        fire_idx(1, 1)
        for b in range(2):
            wait_idx(b)
            gather(b)
            fire_idx(b + 2, b)
            fire_out(b, b)

        def body(j, carry):
            for b in range(2):
                g = 2 * j + b

                wait_idx(b)
                wait_out(b)
                gather(b)

                @pl.when(g + 2 < n_chunks)
                def _():
                    fire_idx(g + 2, b)

                fire_out(g, b)
            return carry

        lax.fori_loop(1, n_pairs, body, 0)
        wait_out(0)
        wait_out(1)

    return k(vectors, flat_idx)


def kernel(x, vectors):
    B, H = x.shape
    D = vectors.shape[1]
    flat = x.reshape(-1).astype(jnp.int32)
    out = _gather(vectors, flat, 1600)
    return out.reshape(B, H, D)
